# baseline (speedup 1.0000x reference)
import jax
import jax.numpy as jnp
from jax import lax
from jax.experimental import pallas as pl
from jax.experimental.pallas import tpu as pltpu
from jax.experimental.pallas import tpu_sc as plsc

_B = 16384
_D = 16
_DH = 8
_NC = 2
_NS = 16
_NW = _NC * _NS
_BPW = _B // _NW
_G = _BPW // 16
_NMAIN = 999936


def _make_half_kernel(h):
    def _half_kernel(
        a_hbm, tail_hbm, idx_hbm, out_hbm, el_v, bas_v, offs, tail_v,
        stage, sem
    ):
        wid = lax.axis_index("s") * _NC + lax.axis_index("c")
        base = wid * _BPW
        pltpu.sync_copy(idx_hbm.at[pl.ds(base, _BPW)], el_v)
        pltpu.sync_copy(tail_hbm, tail_v)

        c127 = jnp.full((16,), 127, jnp.int32)
        cmax = jnp.full((16,), _NMAIN - 1, jnp.int32)
        any_tail = jnp.full((16,), 0, jnp.int32)
        for g in range(_G):
            j0 = g * 16
            el_g = el_v[pl.ds(j0, 16)]
            el_c = lax.min(el_g, cmax)
            any_tail = lax.max(any_tail, lax.sub(el_g, el_c))
            bas_v[pl.ds(j0, 16)] = lax.add(
                lax.shift_left(lax.shift_right_logical(el_c, 7), 10),
                lax.bitwise_and(el_c, c127),
            )
        has_tail = lax.reduce_max(any_tail, (0,)) > 0

        copies = []
        for c in range(_DH):
            ccst = jnp.full((16,), c * 128, jnp.int32)
            for g in range(_G):
                j0 = g * 16
                offs[pl.ds(c * _BPW + j0, 16)] = lax.add(
                    bas_v[pl.ds(j0, 16)], ccst
                )
            copies.append(
                pltpu.async_copy(
                    a_hbm.at[offs.at[pl.ds(c * _BPW, _BPW)]],
                    stage.at[c],
                    sem,
                )
            )
        for cp in copies:
            cp.wait()

        @pl.when(has_tail)
        def _patch_tail():
            cnm = jnp.full((16,), _NMAIN, jnp.int32)
            zero = jnp.full((16,), 0, jnp.int32)
            for g in range(_G):
                j0 = g * 16
                el_g = el_v[pl.ds(j0, 16)]
                is_tail = lax.ge(el_g, cnm)
                toff = lax.shift_left(
                    lax.max(lax.sub(el_g, cnm), zero), 4
                )
                for c in range(_DH):
                    col = jnp.full((16,), h * _DH + c, jnp.int32)
                    vals = stage[c, pl.ds(j0, 16)]
                    tv = plsc.load_gather(tail_v, [lax.add(toff, col)])
                    stage[c, pl.ds(j0, 16)] = lax.select(is_tail, tv, vals)

        pltpu.sync_copy(stage, out_hbm.at[:, pl.ds(base, _BPW)])

    return _half_kernel


def _run_half(h, a_h, tail, el):
    mesh = plsc.VectorSubcoreMesh(core_axis_name="c", subcore_axis_name="s")
    return pl.kernel(
        _make_half_kernel(h),
        mesh=mesh,
        out_type=jax.ShapeDtypeStruct((_DH, _B), jnp.float32),
        scratch_types=[
            pltpu.VMEM((_BPW,), jnp.int32),
            pltpu.VMEM((_BPW,), jnp.int32),
            pltpu.VMEM((_DH * _BPW,), jnp.int32),
            pltpu.VMEM((64 * _D,), jnp.float32),
            pltpu.VMEM((_DH, _BPW), jnp.float32),
            pltpu.SemaphoreType.DMA,
        ],
        compiler_params=pltpu.CompilerParams(
            use_tc_tiling_on_sc=False, needs_layout_passes=False
        ),
        name=f"gather_half{h}",
    )(a_h, tail, el)


@jax.jit
def _gather(a0, a1, tail, el):
    v0 = _run_half(0, a0, tail, el)
    v1 = _run_half(1, a1, tail, el)
    return jnp.concatenate([v0, v1], axis=0).T


def kernel(x, el):
    xt = x.T
    halves = [
        xt[h * _DH : (h + 1) * _DH, :_NMAIN]
        .reshape(_DH, _NMAIN // 128, 128)
        .transpose(1, 0, 2)
        .reshape(-1)
        for h in range(2)
    ]
    tail = x[_NMAIN:].reshape(-1)
    return _gather(halves[0], halves[1], tail, el.astype(jnp.int32))


# speedup vs baseline: 1.1149x; 1.1149x over previous
import jax
import jax.numpy as jnp
from jax import lax
from jax.experimental import pallas as pl
from jax.experimental.pallas import tpu as pltpu
from jax.experimental.pallas import tpu_sc as plsc

_B = 16384
_D = 16
_NC = 2
_NS = 16
_NW = _NC * _NS
_BPW = _B // _NW
_G = _BPW // 16
_NMAIN = 999936
_HALF = 7812 * 1024


def _gather_kernel(
    a_hbm, tail_hbm, idx_hbm, out_hbm, el_v, bas_v, offs, tail_v, stage, sem
):
    wid = lax.axis_index("s") * _NC + lax.axis_index("c")
    base = wid * _BPW
    pltpu.sync_copy(idx_hbm.at[pl.ds(base, _BPW)], el_v)

    c127 = jnp.full((16,), 127, jnp.int32)
    cmax = jnp.full((16,), _NMAIN - 1, jnp.int32)
    any_tail = jnp.full((16,), 0, jnp.int32)
    for g in range(_G):
        j0 = g * 16
        el_g = el_v[pl.ds(j0, 16)]
        el_c = lax.min(el_g, cmax)
        any_tail = lax.max(any_tail, lax.sub(el_g, el_c))
        bas_v[pl.ds(j0, 16)] = lax.add(
            lax.shift_left(lax.shift_right_logical(el_c, 7), 10),
            lax.bitwise_and(el_c, c127),
        )
    has_tail = lax.reduce_max(any_tail, (0,)) > 0

    copies = []
    for c in range(_D):
        ccst = jnp.full((16,), (c // 8) * _HALF + (c % 8) * 128, jnp.int32)
        for g in range(_G):
            j0 = g * 16
            offs[pl.ds(c * _BPW + j0, 16)] = lax.add(
                bas_v[pl.ds(j0, 16)], ccst
            )
        copies.append(
            pltpu.async_copy(
                a_hbm.at[offs.at[pl.ds(c * _BPW, _BPW)]],
                stage.at[c],
                sem,
            )
        )
    for cp in copies:
        cp.wait()

    @pl.when(has_tail)
    def _patch_tail():
        pltpu.sync_copy(tail_hbm, tail_v)
        cnm = jnp.full((16,), _NMAIN, jnp.int32)
        zero = jnp.full((16,), 0, jnp.int32)
        for g in range(_G):
            j0 = g * 16
            el_g = el_v[pl.ds(j0, 16)]
            is_tail = lax.ge(el_g, cnm)
            toff = lax.shift_left(lax.max(lax.sub(el_g, cnm), zero), 4)
            for c in range(_D):
                vals = stage[c, pl.ds(j0, 16)]
                tv = plsc.load_gather(
                    tail_v, [lax.add(toff, jnp.full((16,), c, jnp.int32))]
                )
                stage[c, pl.ds(j0, 16)] = lax.select(is_tail, tv, vals)

    pltpu.sync_copy(stage, out_hbm.at[:, pl.ds(base, _BPW)])


@jax.jit
def _gather(a1, tail, el):
    mesh = plsc.VectorSubcoreMesh(core_axis_name="c", subcore_axis_name="s")
    v = pl.kernel(
        _gather_kernel,
        mesh=mesh,
        out_type=jax.ShapeDtypeStruct((_D, _B), jnp.float32),
        scratch_types=[
            pltpu.VMEM((_BPW,), jnp.int32),
            pltpu.VMEM((_BPW,), jnp.int32),
            pltpu.VMEM((_D * _BPW,), jnp.int32),
            pltpu.VMEM((64 * _D,), jnp.float32),
            pltpu.VMEM((_D, _BPW), jnp.float32),
            pltpu.SemaphoreType.DMA,
        ],
        compiler_params=pltpu.CompilerParams(
            use_tc_tiling_on_sc=False, needs_layout_passes=False
        ),
    )(a1, tail, el)
    return v.T


def kernel(x, el):
    a1 = (
        x.T[:, :_NMAIN]
        .reshape(2, 8, _NMAIN // 128, 128)
        .transpose(0, 2, 1, 3)
        .reshape(-1)
    )
    tail = x[_NMAIN:].reshape(-1)
    return _gather(a1, tail, el.astype(jnp.int32))
